# baseline (speedup 1.0000x reference)
import functools

import jax
import jax.numpy as jnp
from jax import lax
from jax.experimental import pallas as pl
from jax.experimental.pallas import tpu as pltpu
from jax.experimental.pallas import tpu_sc as plsc

NC = 2
NS = 16
NW = NC * NS
CHUNK = 128
NBUF = 2
NSLOT = 4


def _matmul_body(x_ref, w_ref, o_ref):
    o_ref[...] = jnp.dot(x_ref[...], w_ref[...],
                         preferred_element_type=jnp.float32)


def _combine_matmul_body(p_ref, b_ref, w_ref, o_ref):
    h = jnp.maximum(p_ref[0] + p_ref[1] + b_ref[...], 0.0)
    o_ref[...] = jnp.dot(h, w_ref[...], preferred_element_type=jnp.float32)


def _combine_body(p_ref, b_ref, o_ref):
    o_ref[...] = jnp.maximum(p_ref[0] + p_ref[1] + b_ref[...], 0.0)


def _tc_matmul(x, w, block_rows):
    n, d = x.shape
    grid = n // block_rows
    return pl.pallas_call(
        _matmul_body,
        grid=(grid,),
        in_specs=[
            pl.BlockSpec((block_rows, d), lambda i: (i, 0)),
            pl.BlockSpec((d, d), lambda i: (0, 0)),
        ],
        out_specs=pl.BlockSpec((block_rows, d), lambda i: (i, 0)),
        out_shape=jax.ShapeDtypeStruct((n, d), jnp.float32),
    )(x, w)


def _tc_combine_matmul(p, b2d, w, block_rows):
    _, n, d = p.shape
    grid = n // block_rows
    return pl.pallas_call(
        _combine_matmul_body,
        grid=(grid,),
        in_specs=[
            pl.BlockSpec((2, block_rows, d), lambda i: (0, i, 0)),
            pl.BlockSpec((1, d), lambda i: (0, 0)),
            pl.BlockSpec((d, d), lambda i: (0, 0)),
        ],
        out_specs=pl.BlockSpec((block_rows, d), lambda i: (i, 0)),
        out_shape=jax.ShapeDtypeStruct((n, d), jnp.float32),
    )(p, b2d, w)


def _tc_combine(p, b2d, block_rows):
    _, n, d = p.shape
    grid = n // block_rows
    return pl.pallas_call(
        _combine_body,
        grid=(grid,),
        in_specs=[
            pl.BlockSpec((2, block_rows, d), lambda i: (0, i, 0)),
            pl.BlockSpec((1, d), lambda i: (0, 0)),
        ],
        out_specs=pl.BlockSpec((block_rows, d), lambda i: (i, 0)),
        out_shape=jax.ShapeDtypeStruct((n, d), jnp.float32),
    )(p, b2d)


def _make_sc_scatter(n_rows, d, acc_rows, chunks):
    mesh = plsc.VectorSubcoreMesh(core_axis_name="c", subcore_axis_name="s")
    zrows = acc_rows // NS
    ofull = ((n_rows + NS - 1) // NS + 7) // 8 * 8
    olast = n_rows - (NS - 1) * ofull
    assert olast > 0 and olast % 8 == 0

    assert chunks % (2 * NSLOT) == 0
    groups = chunks // NSLOT

    @functools.partial(
        pl.kernel,
        out_type=jax.ShapeDtypeStruct((NC, n_rows, d), jnp.float32),
        mesh=mesh,
        scratch_types=[
            pltpu.VMEM((2, NSLOT, 2, CHUNK), jnp.int32),
            [pltpu.VMEM((CHUNK, d), jnp.float32) for _ in range(NBUF)],
            pltpu.VMEM_SHARED((acc_rows, d), jnp.float32),
            pltpu.SemaphoreType.DMA,
            pltpu.SemaphoreType.DMA,
        ],
    )
    def sc_scatter(support_hbm, idx_hbm, zeros_hbm, out_hbm,
                   idx_v, bufs, acc, gsem, isem):
        c = lax.axis_index("c")
        s = lax.axis_index("s")
        wid = s * NC + c

        pltpu.sync_copy(zeros_hbm.at[pl.ds(s * zrows, zrows)],
                        acc.at[pl.ds(s * zrows, zrows)])
        plsc.subcore_barrier()

        pltpu.async_copy(idx_hbm.at[wid, 0], idx_v.at[0], isem)
        pltpu.async_copy(idx_hbm.at[wid, 1], idx_v.at[1], isem)
        pltpu.make_async_copy(idx_hbm.at[wid, 0], idx_v.at[0], isem).wait()
        for k in range(NBUF):
            pltpu.async_copy(support_hbm.at[idx_v.at[0, k, 0]],
                             bufs[k], gsem)

        def gather_wait(b):
            pltpu.make_async_copy(support_hbm.at[pl.ds(0, CHUNK)],
                                  bufs[b % NBUF], gsem).wait()

        def body(gg, carry):
            for gp in range(2):
                g = gg * 2 + gp
                p, q = gp, 1 - gp
                for b in range(NSLOT):
                    j = g * NSLOT + b
                    gather_wait(b)
                    pltpu.sync_copy(bufs[b % NBUF],
                                    acc.at[idx_v.at[p, b, 1]], add=True)
                    if b == NSLOT - 2:
                        @pl.when(g + 1 < groups)
                        def _():
                            pltpu.make_async_copy(idx_hbm.at[wid, 0],
                                                  idx_v.at[q], isem).wait()
                    nb = b + NBUF
                    if nb < NSLOT:
                        @pl.when(j + NBUF < chunks)
                        def _():
                            pltpu.async_copy(
                                support_hbm.at[idx_v.at[p, nb, 0]],
                                bufs[b % NBUF], gsem)
                    else:
                        @pl.when(j + NBUF < chunks)
                        def _():
                            pltpu.async_copy(
                                support_hbm.at[idx_v.at[q, nb - NSLOT, 0]],
                                bufs[b % NBUF], gsem)
                    if b == NSLOT - 1:
                        @pl.when(g + 2 < groups)
                        def _():
                            pltpu.async_copy(idx_hbm.at[wid, g + 2],
                                             idx_v.at[p], isem)
            return carry

        lax.fori_loop(0, groups // 2, body, 0, unroll=False)
        plsc.subcore_barrier()

        @pl.when(s < NS - 1)
        def _():
            pltpu.sync_copy(acc.at[pl.ds(s * ofull, ofull)],
                            out_hbm.at[c, pl.ds(s * ofull, ofull)])

        @pl.when(s == NS - 1)
        def _():
            pltpu.sync_copy(acc.at[pl.ds((NS - 1) * ofull, olast)],
                            out_hbm.at[c, pl.ds((NS - 1) * ofull, olast)])

    return sc_scatter


def kernel(feature, edge_index, W1, b1, W2, b2, W3, b3):
    n, d = feature.shape
    e = edge_index.shape[1]

    chunks = -(-(-(-e // (NW * CHUNK))) // (2 * NSLOT)) * 2 * NSLOT
    epw = chunks * CHUNK
    e_pad = epw * NW
    acc_rows = -(-(n + 1) // (NS * 8)) * NS * 8

    row = edge_index[0]
    col = edge_index[1]
    pad = e_pad - e
    junk = n + (jnp.arange(pad, dtype=jnp.int32) % (acc_rows - n))
    row_p = jnp.concatenate([row, junk])
    col_p = jnp.concatenate([col, jnp.zeros((pad,), jnp.int32)])
    row3 = row_p.reshape(NW, chunks, CHUNK)
    col3 = col_p.reshape(NW, chunks, CHUNK)
    idx4 = jnp.stack([col3, row3], axis=2).reshape(
        NW, chunks // NSLOT, NSLOT, 2, CHUNK)
    zeros = jnp.zeros((acc_rows, d), jnp.float32)

    sc_scatter = _make_sc_scatter(n, d, acc_rows, chunks)
    block_rows = 2000

    b1_2d = b1.reshape(1, d)
    b2_2d = b2.reshape(1, d)
    b3_2d = b3.reshape(1, d)

    support = _tc_matmul(feature, W1, block_rows)
    p = sc_scatter(support, idx4, zeros)
    support = _tc_combine_matmul(p, b1_2d, W2, block_rows)
    p = sc_scatter(support, idx4, zeros)
    support = _tc_combine_matmul(p, b2_2d, W3, block_rows)
    p = sc_scatter(support, idx4, zeros)
    return _tc_combine(p, b3_2d, block_rows)


# speedup vs baseline: 1.0485x; 1.0485x over previous
import functools

import jax
import jax.numpy as jnp
from jax import lax
from jax.experimental import pallas as pl
from jax.experimental.pallas import tpu as pltpu
from jax.experimental.pallas import tpu_sc as plsc

NC = 2
NS = 16
NW = NC * NS
CHUNK = 128
NBUF = 2
NSLOT = 4


def _matmul_body(x_ref, w_ref, o_ref):
    o_ref[...] = jnp.dot(x_ref[...], w_ref[...],
                         preferred_element_type=jnp.float32)


def _combine_matmul_body(p_ref, b_ref, w_ref, o_ref):
    h = jnp.maximum(p_ref[0] + p_ref[1] + b_ref[...], 0.0)
    o_ref[...] = jnp.dot(h, w_ref[...], preferred_element_type=jnp.float32)


def _combine_body(p_ref, b_ref, o_ref):
    o_ref[...] = jnp.maximum(p_ref[0] + p_ref[1] + b_ref[...], 0.0)


def _tc_matmul(x, w, block_rows):
    n, d = x.shape
    grid = n // block_rows
    return pl.pallas_call(
        _matmul_body,
        grid=(grid,),
        in_specs=[
            pl.BlockSpec((block_rows, d), lambda i: (i, 0)),
            pl.BlockSpec((d, d), lambda i: (0, 0)),
        ],
        out_specs=pl.BlockSpec((block_rows, d), lambda i: (i, 0)),
        out_shape=jax.ShapeDtypeStruct((n, d), jnp.float32),
    )(x, w)


def _tc_combine_matmul(p, b2d, w, block_rows):
    _, n, d = p.shape
    grid = n // block_rows
    return pl.pallas_call(
        _combine_matmul_body,
        grid=(grid,),
        in_specs=[
            pl.BlockSpec((2, block_rows, d), lambda i: (0, i, 0)),
            pl.BlockSpec((1, d), lambda i: (0, 0)),
            pl.BlockSpec((d, d), lambda i: (0, 0)),
        ],
        out_specs=pl.BlockSpec((block_rows, d), lambda i: (i, 0)),
        out_shape=jax.ShapeDtypeStruct((n, d), jnp.float32),
    )(p, b2d, w)


def _tc_combine(p, b2d, block_rows):
    _, n, d = p.shape
    grid = n // block_rows
    return pl.pallas_call(
        _combine_body,
        grid=(grid,),
        in_specs=[
            pl.BlockSpec((2, block_rows, d), lambda i: (0, i, 0)),
            pl.BlockSpec((1, d), lambda i: (0, 0)),
        ],
        out_specs=pl.BlockSpec((block_rows, d), lambda i: (i, 0)),
        out_shape=jax.ShapeDtypeStruct((n, d), jnp.float32),
    )(p, b2d)


def _make_sc_scatter(n_rows, d, acc_rows, chunks):
    mesh = plsc.VectorSubcoreMesh(core_axis_name="c", subcore_axis_name="s")
    zrows = acc_rows // NS
    ofull = ((n_rows + NS - 1) // NS + 7) // 8 * 8
    olast = n_rows - (NS - 1) * ofull
    assert olast > 0 and olast % 8 == 0

    assert chunks % 2 == 0

    @functools.partial(
        pl.kernel,
        out_type=jax.ShapeDtypeStruct((NC, n_rows, d), jnp.float32),
        mesh=mesh,
        scratch_types=[
            pltpu.VMEM((chunks, CHUNK), jnp.int32),
            pltpu.VMEM((2, 2, CHUNK), jnp.int32),
            [pltpu.VMEM((CHUNK, d), jnp.float32) for _ in range(2)],
            pltpu.VMEM_SHARED((acc_rows, d), jnp.float32),
            pltpu.SemaphoreType.DMA,
        ],
    )
    def sc_scatter(support_hbm, idx_hbm, zeros_hbm, out_hbm,
                   pk_v, dec_v, bufs, acc, gsem):
        c = lax.axis_index("c")
        s = lax.axis_index("s")
        wid = s * NC + c

        pltpu.sync_copy(idx_hbm.at[wid], pk_v)
        pltpu.sync_copy(zeros_hbm.at[pl.ds(s * zrows, zrows)],
                        acc.at[pl.ds(s * zrows, zrows)])
        plsc.subcore_barrier()

        def decode(j, slot):
            for t in range(CHUNK // 16):
                v = pk_v[j, pl.ds(16 * t, 16)]
                dec_v[slot, 0, pl.ds(16 * t, 16)] = lax.bitwise_and(
                    v, 0xFFFF)
                dec_v[slot, 1, pl.ds(16 * t, 16)] = lax.shift_right_logical(
                    v, 16)

        for b in range(2):
            decode(b, b)
            pltpu.async_copy(support_hbm.at[dec_v.at[b, 0]], bufs[b], gsem)

        def body(g, carry):
            for b in range(2):
                j = g * 2 + b
                pltpu.make_async_copy(support_hbm.at[pl.ds(0, CHUNK)],
                                      bufs[b], gsem).wait()
                pltpu.sync_copy(bufs[b], acc.at[dec_v.at[b, 1]], add=True)

                @pl.when(j + 2 < chunks)
                def _():
                    decode(j + 2, b)
                    pltpu.async_copy(support_hbm.at[dec_v.at[b, 0]],
                                     bufs[b], gsem)
            return carry

        lax.fori_loop(0, chunks // 2, body, 0, unroll=False)
        plsc.subcore_barrier()

        @pl.when(s < NS - 1)
        def _():
            pltpu.sync_copy(acc.at[pl.ds(s * ofull, ofull)],
                            out_hbm.at[c, pl.ds(s * ofull, ofull)])

        @pl.when(s == NS - 1)
        def _():
            pltpu.sync_copy(acc.at[pl.ds((NS - 1) * ofull, olast)],
                            out_hbm.at[c, pl.ds((NS - 1) * ofull, olast)])

    return sc_scatter


def kernel(feature, edge_index, W1, b1, W2, b2, W3, b3):
    n, d = feature.shape
    e = edge_index.shape[1]

    chunks = -(-(-(-e // (NW * CHUNK))) // 2) * 2
    epw = chunks * CHUNK
    e_pad = epw * NW
    acc_rows = -(-(n + 1) // (NS * 8)) * NS * 8

    row = edge_index[0]
    col = edge_index[1]
    pad = e_pad - e
    junk = n + (jnp.arange(pad, dtype=jnp.int32) % (acc_rows - n))
    row_p = jnp.concatenate([row, junk])
    col_p = jnp.concatenate([col, jnp.zeros((pad,), jnp.int32)])
    row3 = row_p.reshape(NW, chunks, CHUNK)
    col3 = col_p.reshape(NW, chunks, CHUNK)
    packed = (col3 | (row3 << 16)).astype(jnp.int32)
    zeros = jnp.zeros((acc_rows, d), jnp.float32)

    sc_scatter = _make_sc_scatter(n, d, acc_rows, chunks)
    block_rows = 2000

    b1_2d = b1.reshape(1, d)
    b2_2d = b2.reshape(1, d)
    b3_2d = b3.reshape(1, d)

    support = _tc_matmul(feature, W1, block_rows)
    p = sc_scatter(support, packed, zeros)
    support = _tc_combine_matmul(p, b1_2d, W2, block_rows)
    p = sc_scatter(support, packed, zeros)
    support = _tc_combine_matmul(p, b2_2d, W3, block_rows)
    p = sc_scatter(support, packed, zeros)
    return _tc_combine(p, b3_2d, block_rows)


# speedup vs baseline: 1.4010x; 1.3362x over previous
import functools

import jax
import jax.numpy as jnp
from jax import lax
from jax.experimental import pallas as pl
from jax.experimental.pallas import tpu as pltpu
from jax.experimental.pallas import tpu_sc as plsc

NC = 2
NS = 16
NW = NC * NS
CHUNK = 128
NBUF = 2
NSLOT = 4


def _matmul_body(x_ref, w_ref, o_ref):
    o_ref[...] = jnp.dot(x_ref[...], w_ref[...],
                         preferred_element_type=jnp.float32)


def _combine_matmul_body(p_ref, b_ref, w_ref, o_ref):
    h = jnp.maximum(p_ref[0] + p_ref[1] + b_ref[...], 0.0)
    o_ref[...] = jnp.dot(h, w_ref[...], preferred_element_type=jnp.float32)


def _combine_body(p_ref, b_ref, o_ref):
    o_ref[...] = jnp.maximum(p_ref[0] + p_ref[1] + b_ref[...], 0.0)


def _tc_matmul(x, w, block_rows):
    n, d = x.shape
    grid = n // block_rows
    return pl.pallas_call(
        _matmul_body,
        grid=(grid,),
        in_specs=[
            pl.BlockSpec((block_rows, d), lambda i: (i, 0)),
            pl.BlockSpec((d, d), lambda i: (0, 0)),
        ],
        out_specs=pl.BlockSpec((block_rows, d), lambda i: (i, 0)),
        out_shape=jax.ShapeDtypeStruct((n, d), jnp.float32),
    )(x, w)


def _tc_combine_matmul(p, b2d, w, block_rows):
    _, n, d = p.shape
    grid = n // block_rows
    return pl.pallas_call(
        _combine_matmul_body,
        grid=(grid,),
        in_specs=[
            pl.BlockSpec((2, block_rows, d), lambda i: (0, i, 0)),
            pl.BlockSpec((1, d), lambda i: (0, 0)),
            pl.BlockSpec((d, d), lambda i: (0, 0)),
        ],
        out_specs=pl.BlockSpec((block_rows, d), lambda i: (i, 0)),
        out_shape=jax.ShapeDtypeStruct((n, d), jnp.float32),
    )(p, b2d, w)


def _tc_combine(p, b2d, block_rows):
    _, n, d = p.shape
    grid = n // block_rows
    return pl.pallas_call(
        _combine_body,
        grid=(grid,),
        in_specs=[
            pl.BlockSpec((2, block_rows, d), lambda i: (0, i, 0)),
            pl.BlockSpec((1, d), lambda i: (0, 0)),
        ],
        out_specs=pl.BlockSpec((block_rows, d), lambda i: (i, 0)),
        out_shape=jax.ShapeDtypeStruct((n, d), jnp.float32),
    )(p, b2d)


def _make_sc_scatter(n_rows, d, acc_rows, chunks):
    mesh = plsc.VectorSubcoreMesh(core_axis_name="c", subcore_axis_name="s")
    zrows = acc_rows // NS
    ofull = ((n_rows + NS - 1) // NS + 7) // 8 * 8
    olast = n_rows - (NS - 1) * ofull
    assert olast > 0 and olast % 8 == 0

    @functools.partial(
        pl.kernel,
        out_type=jax.ShapeDtypeStruct((NC, n_rows, d), jnp.float32),
        mesh=mesh,
        scratch_types=[
            pltpu.VMEM((chunks, CHUNK), jnp.int32),
            pltpu.VMEM((chunks, CHUNK), jnp.int32),
            pltpu.VMEM((CHUNK, d), jnp.float32),
            pltpu.VMEM_SHARED((acc_rows, d), jnp.float32),
            pltpu.SemaphoreType.DMA,
        ],
    )
    def sc_scatter(support_hbm, idx_hbm, zeros_hbm, out_hbm,
                   col_v, row_v, buf, acc, sem):
        c = lax.axis_index("c")
        s = lax.axis_index("s")
        wid = s * NC + c

        pltpu.sync_copy(idx_hbm.at[wid, 0], col_v)
        pltpu.sync_copy(idx_hbm.at[wid, 1], row_v)
        pltpu.sync_copy(zeros_hbm.at[pl.ds(s * zrows, zrows)],
                        acc.at[pl.ds(s * zrows, zrows)])
        plsc.subcore_barrier()

        def body(j, carry):
            pltpu.async_copy(support_hbm.at[col_v.at[j]], buf, sem).wait()
            pltpu.sync_copy(buf, acc.at[row_v.at[j]], add=True)
            return carry

        lax.fori_loop(0, chunks, body, 0, unroll=2)
        plsc.subcore_barrier()

        @pl.when(s < NS - 1)
        def _():
            pltpu.sync_copy(acc.at[pl.ds(s * ofull, ofull)],
                            out_hbm.at[c, pl.ds(s * ofull, ofull)])

        @pl.when(s == NS - 1)
        def _():
            pltpu.sync_copy(acc.at[pl.ds((NS - 1) * ofull, olast)],
                            out_hbm.at[c, pl.ds((NS - 1) * ofull, olast)])

    return sc_scatter


def kernel(feature, edge_index, W1, b1, W2, b2, W3, b3):
    n, d = feature.shape
    e = edge_index.shape[1]

    chunks = -(-e // (NW * CHUNK))
    epw = chunks * CHUNK
    e_pad = epw * NW
    acc_rows = -(-(n + 1) // (NS * 8)) * NS * 8

    row = edge_index[0]
    col = edge_index[1]
    pad = e_pad - e
    junk = n + (jnp.arange(pad, dtype=jnp.int32) % (acc_rows - n))
    row_p = jnp.concatenate([row, junk])
    col_p = jnp.concatenate([col, jnp.zeros((pad,), jnp.int32)])
    row3 = row_p.reshape(NW, chunks, CHUNK)
    col3 = col_p.reshape(NW, chunks, CHUNK)
    idx4 = jnp.stack([col3, row3], axis=1)
    zeros = jnp.zeros((acc_rows, d), jnp.float32)

    sc_scatter = _make_sc_scatter(n, d, acc_rows, chunks)
    block_rows = 2000

    b1_2d = b1.reshape(1, d)
    b2_2d = b2.reshape(1, d)
    b3_2d = b3.reshape(1, d)

    support = _tc_matmul(feature, W1, block_rows)
    p = sc_scatter(support, idx4, zeros)
    support = _tc_combine_matmul(p, b1_2d, W2, block_rows)
    p = sc_scatter(support, idx4, zeros)
    support = _tc_combine_matmul(p, b2_2d, W3, block_rows)
    p = sc_scatter(support, idx4, zeros)
    return _tc_combine(p, b3_2d, block_rows)


# speedup vs baseline: 1.4220x; 1.0150x over previous
import functools

import jax
import jax.numpy as jnp
from jax import lax
from jax.experimental import pallas as pl
from jax.experimental.pallas import tpu as pltpu
from jax.experimental.pallas import tpu_sc as plsc

NC = 2
NS = 16
NW = NC * NS
CHUNK = 128
NBUF = 2
NSLOT = 4


def _matmul_body(x_ref, w_ref, o_ref):
    o_ref[...] = jnp.dot(x_ref[...], w_ref[...],
                         preferred_element_type=jnp.float32)


def _combine_matmul_body(p_ref, b_ref, w_ref, o_ref):
    h = jnp.maximum(p_ref[0] + p_ref[1] + b_ref[...], 0.0)
    o_ref[...] = jnp.dot(h, w_ref[...], preferred_element_type=jnp.float32)


def _combine_body(p_ref, b_ref, o_ref):
    o_ref[...] = jnp.maximum(p_ref[0] + p_ref[1] + b_ref[...], 0.0)


def _tc_matmul(x, w, block_rows):
    n, d = x.shape
    grid = n // block_rows
    return pl.pallas_call(
        _matmul_body,
        grid=(grid,),
        in_specs=[
            pl.BlockSpec((block_rows, d), lambda i: (i, 0)),
            pl.BlockSpec((d, d), lambda i: (0, 0)),
        ],
        out_specs=pl.BlockSpec((block_rows, d), lambda i: (i, 0)),
        out_shape=jax.ShapeDtypeStruct((n, d), jnp.float32),
    )(x, w)


def _tc_combine_matmul(p, b2d, w, block_rows):
    _, n, d = p.shape
    grid = n // block_rows
    return pl.pallas_call(
        _combine_matmul_body,
        grid=(grid,),
        in_specs=[
            pl.BlockSpec((2, block_rows, d), lambda i: (0, i, 0)),
            pl.BlockSpec((1, d), lambda i: (0, 0)),
            pl.BlockSpec((d, d), lambda i: (0, 0)),
        ],
        out_specs=pl.BlockSpec((block_rows, d), lambda i: (i, 0)),
        out_shape=jax.ShapeDtypeStruct((n, d), jnp.float32),
    )(p, b2d, w)


def _tc_combine(p, b2d, block_rows):
    _, n, d = p.shape
    grid = n // block_rows
    return pl.pallas_call(
        _combine_body,
        grid=(grid,),
        in_specs=[
            pl.BlockSpec((2, block_rows, d), lambda i: (0, i, 0)),
            pl.BlockSpec((1, d), lambda i: (0, 0)),
        ],
        out_specs=pl.BlockSpec((block_rows, d), lambda i: (i, 0)),
        out_shape=jax.ShapeDtypeStruct((n, d), jnp.float32),
    )(p, b2d)


def _make_sc_scatter(n_rows, d, acc_rows, chunks):
    mesh = plsc.VectorSubcoreMesh(core_axis_name="c", subcore_axis_name="s")
    zrows = acc_rows // NS
    ofull = ((n_rows + NS - 1) // NS + 7) // 8 * 8
    olast = n_rows - (NS - 1) * ofull
    assert olast > 0 and olast % 8 == 0

    @functools.partial(
        pl.kernel,
        out_type=jax.ShapeDtypeStruct((NC, n_rows, d), jnp.float32),
        mesh=mesh,
        scratch_types=[
            pltpu.VMEM((chunks, CHUNK), jnp.int32),
            pltpu.VMEM((chunks, CHUNK), jnp.int32),
            pltpu.VMEM((CHUNK, d), jnp.float32),
            pltpu.VMEM_SHARED((acc_rows, d), jnp.float32),
            pltpu.SemaphoreType.DMA,
        ],
    )
    def sc_scatter(support_hbm, idx_hbm, zeros_hbm, out_hbm,
                   col_v, row_v, buf, acc, sem):
        c = lax.axis_index("c")
        s = lax.axis_index("s")
        wid = s * NC + c

        pltpu.sync_copy(idx_hbm.at[wid, 0], col_v)
        pltpu.sync_copy(idx_hbm.at[wid, 1], row_v)
        pltpu.sync_copy(zeros_hbm.at[pl.ds(s * zrows, zrows)],
                        acc.at[pl.ds(s * zrows, zrows)])
        plsc.subcore_barrier()

        def body(j, carry):
            pltpu.async_copy(support_hbm.at[col_v.at[j]], buf, sem).wait()
            pltpu.sync_copy(buf, acc.at[row_v.at[j]], add=True)
            return carry

        lax.fori_loop(0, chunks, body, 0, unroll=False)
        plsc.subcore_barrier()

        @pl.when(s < NS - 1)
        def _():
            pltpu.sync_copy(acc.at[pl.ds(s * ofull, ofull)],
                            out_hbm.at[c, pl.ds(s * ofull, ofull)])

        @pl.when(s == NS - 1)
        def _():
            pltpu.sync_copy(acc.at[pl.ds((NS - 1) * ofull, olast)],
                            out_hbm.at[c, pl.ds((NS - 1) * ofull, olast)])

    return sc_scatter


def kernel(feature, edge_index, W1, b1, W2, b2, W3, b3):
    n, d = feature.shape
    e = edge_index.shape[1]

    chunks = -(-e // (NW * CHUNK))
    epw = chunks * CHUNK
    e_pad = epw * NW
    acc_rows = -(-(n + 1) // (NS * 8)) * NS * 8

    row = edge_index[0]
    col = edge_index[1]
    pad = e_pad - e
    junk = n + (jnp.arange(pad, dtype=jnp.int32) % (acc_rows - n))
    row_p = jnp.concatenate([row, junk])
    col_p = jnp.concatenate([col, jnp.zeros((pad,), jnp.int32)])
    row3 = row_p.reshape(NW, chunks, CHUNK)
    col3 = col_p.reshape(NW, chunks, CHUNK)
    idx4 = jnp.stack([col3, row3], axis=1)
    zeros = jnp.zeros((acc_rows, d), jnp.float32)

    sc_scatter = _make_sc_scatter(n, d, acc_rows, chunks)
    block_rows = 2000

    b1_2d = b1.reshape(1, d)
    b2_2d = b2.reshape(1, d)
    b3_2d = b3.reshape(1, d)

    support = _tc_matmul(feature, W1, block_rows)
    p = sc_scatter(support, idx4, zeros)
    support = _tc_combine_matmul(p, b1_2d, W2, block_rows)
    p = sc_scatter(support, idx4, zeros)
    support = _tc_combine_matmul(p, b2_2d, W3, block_rows)
    p = sc_scatter(support, idx4, zeros)
    return _tc_combine(p, b3_2d, block_rows)


# speedup vs baseline: 1.4459x; 1.0168x over previous
import functools

import jax
import jax.numpy as jnp
from jax import lax
from jax.experimental import pallas as pl
from jax.experimental.pallas import tpu as pltpu
from jax.experimental.pallas import tpu_sc as plsc

NC = 2
NS = 16
NW = NC * NS
CHUNK = 128


def _matmul_body(x_ref, w_ref, o_ref):
    o_ref[...] = jnp.dot(x_ref[...], w_ref[...],
                         preferred_element_type=jnp.float32)


def _combine_matmul_body(p_ref, b_ref, w_ref, o_ref):
    h = jnp.maximum(p_ref[0] + p_ref[1] + b_ref[...], 0.0)
    o_ref[...] = jnp.dot(h, w_ref[...], preferred_element_type=jnp.float32)


def _combine_body(p_ref, b_ref, o_ref):
    o_ref[...] = jnp.maximum(p_ref[0] + p_ref[1] + b_ref[...], 0.0)


def _tc_matmul(x, w, block_rows):
    n, d = x.shape
    grid = n // block_rows
    return pl.pallas_call(
        _matmul_body,
        grid=(grid,),
        in_specs=[
            pl.BlockSpec((block_rows, d), lambda i: (i, 0)),
            pl.BlockSpec((d, d), lambda i: (0, 0)),
        ],
        out_specs=pl.BlockSpec((block_rows, d), lambda i: (i, 0)),
        out_shape=jax.ShapeDtypeStruct((n, d), jnp.float32),
    )(x, w)


def _tc_combine_matmul(p, b2d, w, block_rows):
    _, n, d = p.shape
    grid = n // block_rows
    return pl.pallas_call(
        _combine_matmul_body,
        grid=(grid,),
        in_specs=[
            pl.BlockSpec((2, block_rows, d), lambda i: (0, i, 0)),
            pl.BlockSpec((1, d), lambda i: (0, 0)),
            pl.BlockSpec((d, d), lambda i: (0, 0)),
        ],
        out_specs=pl.BlockSpec((block_rows, d), lambda i: (i, 0)),
        out_shape=jax.ShapeDtypeStruct((n, d), jnp.float32),
    )(p, b2d, w)


def _tc_combine(p, b2d, block_rows):
    _, n, d = p.shape
    grid = n // block_rows
    return pl.pallas_call(
        _combine_body,
        grid=(grid,),
        in_specs=[
            pl.BlockSpec((2, block_rows, d), lambda i: (0, i, 0)),
            pl.BlockSpec((1, d), lambda i: (0, 0)),
        ],
        out_specs=pl.BlockSpec((block_rows, d), lambda i: (i, 0)),
        out_shape=jax.ShapeDtypeStruct((n, d), jnp.float32),
    )(p, b2d)


def _make_sc_scatter(n_rows, d, acc_rows, chunks):
    mesh = plsc.VectorSubcoreMesh(core_axis_name="c", subcore_axis_name="s")
    zrows = acc_rows // NS
    ofull = ((n_rows + NS - 1) // NS + 7) // 8 * 8
    olast = n_rows - (NS - 1) * ofull
    assert olast > 0 and olast % 8 == 0

    @functools.partial(
        pl.kernel,
        out_type=jax.ShapeDtypeStruct((NC, n_rows, d), jnp.float32),
        mesh=mesh,
        scratch_types=[
            pltpu.VMEM((chunks, CHUNK), jnp.int32),
            pltpu.VMEM((chunks, CHUNK), jnp.int32),
            pltpu.VMEM((CHUNK, d), jnp.float32),
            pltpu.VMEM_SHARED((acc_rows, d), jnp.float32),
            pltpu.SemaphoreType.DMA,
        ],
    )
    def sc_scatter(support_hbm, idx_hbm, zeros_hbm, out_hbm,
                   col_v, row_v, buf, acc, sem):
        c = lax.axis_index("c")
        s = lax.axis_index("s")
        wid = s * NC + c

        pltpu.sync_copy(idx_hbm.at[wid, 0], col_v)
        pltpu.sync_copy(idx_hbm.at[wid, 1], row_v)
        pltpu.sync_copy(zeros_hbm.at[pl.ds(s * zrows, zrows)],
                        acc.at[pl.ds(s * zrows, zrows)])
        plsc.subcore_barrier()

        def body(j, carry):
            pltpu.async_copy(support_hbm.at[col_v.at[j]], buf, sem).wait()
            pltpu.sync_copy(buf, acc.at[row_v.at[j]], add=True)
            return carry

        lax.fori_loop(0, chunks, body, 0, unroll=False)
        plsc.subcore_barrier()

        @pl.when(s < NS - 1)
        def _():
            pltpu.sync_copy(acc.at[pl.ds(s * ofull, ofull)],
                            out_hbm.at[c, pl.ds(s * ofull, ofull)])

        @pl.when(s == NS - 1)
        def _():
            pltpu.sync_copy(acc.at[pl.ds((NS - 1) * ofull, olast)],
                            out_hbm.at[c, pl.ds((NS - 1) * ofull, olast)])

    return sc_scatter


def kernel(feature, edge_index, W1, b1, W2, b2, W3, b3):
    n, d = feature.shape
    e = edge_index.shape[1]

    chunks = -(-e // (NW * CHUNK))
    epw = chunks * CHUNK
    e_pad = epw * NW
    acc_rows = -(-(n + 1) // (NS * 8)) * NS * 8

    row = edge_index[0]
    col = edge_index[1]
    pad = e_pad - e
    junk = n + (jnp.arange(pad, dtype=jnp.int32) % (acc_rows - n))
    row_p = jnp.concatenate([row, junk])
    col_p = jnp.concatenate([col, jnp.zeros((pad,), jnp.int32)])
    row3 = row_p.reshape(NW, chunks, CHUNK)
    col3 = col_p.reshape(NW, chunks, CHUNK)
    idx4 = jnp.stack([col3, row3], axis=1)
    zeros = jnp.zeros((acc_rows, d), jnp.float32)

    sc_scatter = _make_sc_scatter(n, d, acc_rows, chunks)
    block_rows = 2000 if n % 2000 == 0 else n

    b1_2d = b1.reshape(1, d)
    b2_2d = b2.reshape(1, d)
    b3_2d = b3.reshape(1, d)

    support = _tc_matmul(feature, W1, block_rows)
    p = sc_scatter(support, idx4, zeros)
    support = _tc_combine_matmul(p, b1_2d, W2, block_rows)
    p = sc_scatter(support, idx4, zeros)
    support = _tc_combine_matmul(p, b2_2d, W3, block_rows)
    p = sc_scatter(support, idx4, zeros)
    return _tc_combine(p, b3_2d, block_rows)
